# baseline (speedup 1.0000x reference)
import functools

import jax
import jax.numpy as jnp
from jax import lax
from jax.experimental import pallas as pl
from jax.experimental.pallas import tpu as pltpu
from jax.experimental.pallas import tpu_sc as plsc

E = 8
K = 2
D = 768
F = 1536
T = 2048
S = T * K
BM = 128
NB = S // BM + E
SPAD = NB * BM
NC, NS = 2, 16
NW = NC * NS
LPW = S // NW
TPW = T // NW

_LANES = 128
_NEG = -1e30


def _route_body(x_ref, rw_ref, pos0_ref, pos1_ref, p0_ref, p1_ref, blk_ref):
    x = x_ref[...]
    rw = rw_ref[...]
    logits = lax.dot_general(x, rw, (((1,), (1,)), ((), ())),
                             preferred_element_type=jnp.float32)
    logits = jnp.concatenate(
        [logits, jnp.full((T, 16 - E), _NEG, jnp.float32)], axis=1)
    col = lax.broadcasted_iota(jnp.int32, (T, 16), 1)
    valid = col < E
    lm = jnp.where(valid, logits, _NEG)
    mx = jnp.max(lm, axis=1, keepdims=True)
    ex = jnp.where(valid, jnp.exp(lm - mx), 0.0)
    probs = ex / jnp.sum(ex, axis=1, keepdims=True)

    p0 = jnp.max(probs, axis=1, keepdims=True)
    i0 = jnp.min(jnp.where(probs == p0, col, 127), axis=1, keepdims=True)
    probs1 = jnp.where(col == i0, 0.0, probs)
    p1 = jnp.max(probs1, axis=1, keepdims=True)
    i1 = jnp.min(jnp.where(probs1 == p1, col, 127), axis=1, keepdims=True)

    m = ((col == 2 * i0) | (col == 2 * i1 + 1)).astype(jnp.float32)

    incl = m
    sh = 1
    while sh < T:
        shifted = jnp.concatenate(
            [jnp.zeros((sh, 16), jnp.float32), incl[: T - sh, :]], axis=0)
        incl = incl + shifted
        sh *= 2
    excl = incl - m
    tot = incl[T - 1 : T, :]

    col1 = lax.broadcasted_iota(jnp.int32, (1, 16), 1)
    starts = []
    counts = []
    acc = jnp.int32(0)
    for e in range(E):
        ce = jnp.sum(jnp.where((col1 == 2 * e) | (col1 == 2 * e + 1), tot, 0.0))
        ce = ce.astype(jnp.int32)
        counts.append(ce)
        starts.append(acc)
        acc = acc + ((ce + BM - 1) // BM) * BM
    total_pad = acc

    off = jnp.zeros((1, 16), jnp.float32)
    for e in range(E):
        off = off + jnp.where((col1 == 2 * e) | (col1 == 2 * e + 1),
                              starts[e].astype(jnp.float32), 0.0)
    tot_even = jnp.where(col1 % 2 == 0, tot, 0.0)
    off = off + jnp.concatenate(
        [jnp.zeros((1, 1), jnp.float32), tot_even[:, :15]], axis=1)

    slotpos = excl + off
    pos0 = jnp.sum(jnp.where(col == 2 * i0, slotpos, 0.0), axis=1, keepdims=True)
    pos1 = jnp.sum(jnp.where(col == 2 * i1 + 1, slotpos, 0.0), axis=1, keepdims=True)
    pos0_ref[...] = jnp.reshape(pos0, (T // _LANES, _LANES)).astype(jnp.int32)
    pos1_ref[...] = jnp.reshape(pos1, (T // _LANES, _LANES)).astype(jnp.int32)
    p0_ref[...] = jnp.broadcast_to(p0, (T, 16))
    p1_ref[...] = jnp.broadcast_to(p1, (T, 16))

    kb = lax.broadcasted_iota(jnp.int32, (1, _LANES), 1)
    row_start = kb * BM
    eb = jnp.full((1, _LANES), -1, jnp.int32)
    match_start = jnp.zeros((1, _LANES), jnp.bool_)
    for e in range(E):
        eb = eb + (row_start >= starts[e]).astype(jnp.int32)
    blkrow = jnp.where(row_start < total_pad, eb, -1)
    for e in range(E):
        match_start = match_start | (row_start == starts[e])
    isfrow = ((blkrow >= 0) & match_start).astype(jnp.int32)

    active = [counts[e] > 0 for e in range(E)]
    na = jnp.int32(-1)
    next_of = [None] * E
    for e in reversed(range(E)):
        next_of[e] = na
        na = jnp.where(active[e], jnp.int32(e), na)
    rank = []
    r = jnp.int32(0)
    for e in range(E):
        rank.append(r)
        r = r + active[e].astype(jnp.int32)

    frow = jnp.full((1, _LANES), -1, jnp.int32)
    ringrow = jnp.zeros((1, _LANES), jnp.int32)
    for e in range(E):
        sel = blkrow == e
        frow = jnp.where(sel & (isfrow == 1), next_of[e], frow)
        ringrow = jnp.where(sel, rank[e] & 1, ringrow)

    blk_ref[0:1, :] = blkrow
    blk_ref[1:2, :] = isfrow
    blk_ref[2:3, :] = frow
    blk_ref[3:4, :] = ringrow
    blk_ref[4:5, :] = jnp.full((1, _LANES), 1, jnp.int32) * (total_pad // BM - 1)
    blk_ref[5:8, :] = jnp.zeros((3, _LANES), jnp.int32)


def _route(x, rw_pad):
    return pl.pallas_call(
        _route_body,
        out_shape=(
            jax.ShapeDtypeStruct((T // _LANES, _LANES), jnp.int32),
            jax.ShapeDtypeStruct((T // _LANES, _LANES), jnp.int32),
            jax.ShapeDtypeStruct((T, 16), jnp.float32),
            jax.ShapeDtypeStruct((T, 16), jnp.float32),
            jax.ShapeDtypeStruct((8, _LANES), jnp.int32),
        ),
    )(x, rw_pad)


@functools.lru_cache(maxsize=None)
def _sc_mesh():
    return plsc.VectorSubcoreMesh(
        core_axis_name="c", subcore_axis_name="s", num_cores=NC, num_subcores=NS)


_NCHUNK = 8
_CROWS = LPW // _NCHUNK


@functools.lru_cache(maxsize=None)
def _dispatch_kernel():
    @functools.partial(
        pl.kernel,
        out_type=jax.ShapeDtypeStruct((SPAD, D), jnp.float32),
        mesh=_sc_mesh(),
        scratch_types=[
            pltpu.VMEM((_NCHUNK, _CROWS), jnp.int32),
            pltpu.VMEM((LPW, D), jnp.float32),
            [pltpu.SemaphoreType.DMA] * _NCHUNK,
            pltpu.SemaphoreType.DMA,
        ],
    )
    def _dispatch(x_hbm, pos0_hbm, pos1_hbm, xs_hbm,
                  dst_v, rows_v, sems_l, sem_s):
        cid = lax.axis_index("c")
        sid = lax.axis_index("s")
        tok_base = sid * LPW

        loads = [
            pltpu.async_copy(
                x_hbm.at[pl.ds(tok_base + c * _CROWS, _CROWS)],
                rows_v.at[pl.ds(c * _CROWS, _CROWS)], sems_l[c])
            for c in range(_NCHUNK)
        ]

        for c in range(_NCHUNK):
            @pl.when(cid == 0)
            def _():
                pltpu.sync_copy(
                    pos0_hbm.at[sid, pl.ds(c * _CROWS, _CROWS)], dst_v.at[c])

            @pl.when(cid == 1)
            def _():
                pltpu.sync_copy(
                    pos1_hbm.at[sid, pl.ds(c * _CROWS, _CROWS)], dst_v.at[c])

        scats = []
        for c in range(_NCHUNK):
            loads[c].wait()
            scats.append(pltpu.async_copy(
                rows_v.at[pl.ds(c * _CROWS, _CROWS)],
                xs_hbm.at[dst_v.at[c]], sem_s))
        for cp in scats:
            cp.wait()

    return _dispatch


def _expert_body(meta_ref, xs_ref, win_hbm, bin_ref, wout_hbm, bout_ref, y_ref,
                 win_v, wout_v, semw):
    b = pl.program_id(0)
    e = meta_ref[0, b]
    isf = meta_ref[1, b]
    fe = meta_ref[2, b]
    ring = meta_ref[3, b]

    @pl.when(b == 0)
    def _():
        e0 = meta_ref[0, 0]
        pltpu.make_async_copy(win_hbm.at[e0], win_v.at[0], semw.at[0]).start()
        pltpu.make_async_copy(wout_hbm.at[e0], wout_v.at[0], semw.at[1]).start()

    @pl.when(fe >= 0)
    def _():
        nring = 1 - ring
        pltpu.make_async_copy(
            win_hbm.at[fe], win_v.at[nring], semw.at[2 * nring]).start()
        pltpu.make_async_copy(
            wout_hbm.at[fe], wout_v.at[nring], semw.at[2 * nring + 1]).start()

    @pl.when(isf == 1)
    def _():
        pltpu.make_async_copy(
            win_hbm.at[e], win_v.at[ring], semw.at[2 * ring]).wait()
        pltpu.make_async_copy(
            wout_hbm.at[e], wout_v.at[ring], semw.at[2 * ring + 1]).wait()

    @pl.when(e >= 0)
    def _():
        xb = xs_ref[...]
        h = jnp.dot(xb, win_v[ring], preferred_element_type=jnp.float32)
        h = jnp.maximum(h + bin_ref[0], 0.0)
        y = jnp.dot(h, wout_v[ring], preferred_element_type=jnp.float32)
        y_ref[...] = y + bout_ref[0]


def _experts(meta, xs, w_in, b_in, w_out, b_out):
    def eidx(b, blk):
        return jnp.where(blk[0, b] < 0, E - 1, blk[0, b])

    def bclamp(b, blk):
        return jnp.where(blk[0, b] < 0, blk[4, 0], b)

    grid_spec = pltpu.PrefetchScalarGridSpec(
        num_scalar_prefetch=1,
        grid=(NB,),
        in_specs=[
            pl.BlockSpec((BM, D), lambda b, blk: (bclamp(b, blk), 0)),
            pl.BlockSpec(memory_space=pl.ANY),
            pl.BlockSpec((1, 1, F), lambda b, blk: (eidx(b, blk), 0, 0)),
            pl.BlockSpec(memory_space=pl.ANY),
            pl.BlockSpec((1, 1, D), lambda b, blk: (eidx(b, blk), 0, 0)),
        ],
        out_specs=pl.BlockSpec((BM, D), lambda b, blk: (bclamp(b, blk), 0)),
        scratch_shapes=[
            pltpu.VMEM((2, D, F), jnp.float32),
            pltpu.VMEM((2, F, D), jnp.float32),
            pltpu.SemaphoreType.DMA((4,)),
        ],
    )
    return pl.pallas_call(
        _expert_body,
        grid_spec=grid_spec,
        out_shape=jax.ShapeDtypeStruct((SPAD, D), jnp.float32),
    )(meta, xs, w_in, b_in.reshape(E, 1, F), w_out, b_out.reshape(E, 1, D))


@functools.lru_cache(maxsize=None)
def _combine_kernel():
    @functools.partial(
        pl.kernel,
        out_type=jax.ShapeDtypeStruct((T, D), jnp.float32),
        mesh=_sc_mesh(),
        scratch_types=[
            pltpu.VMEM((TPW,), jnp.int32),
            pltpu.VMEM((TPW,), jnp.int32),
            pltpu.VMEM((TPW, 16), jnp.float32),
            pltpu.VMEM((TPW, 16), jnp.float32),
            pltpu.VMEM((TPW, D), jnp.float32),
            pltpu.VMEM((TPW, D), jnp.float32),
            [pltpu.SemaphoreType.DMA] * 2,
            pltpu.SemaphoreType.DMA,
        ],
    )
    def _combine(y_hbm, pos0_hbm, pos1_hbm, p0_hbm, p1_hbm, out_hbm,
                 idx0_v, idx1_v, pb0_v, pb1_v, rows0_v, rows1_v, semg, sem_s):
        cid = lax.axis_index("c")
        sid = lax.axis_index("s")
        wid = cid * NS + sid
        tb = wid * TPW
        prow = wid // 2
        pcol = (wid % 2) * TPW
        pltpu.sync_copy(pos0_hbm.at[prow, pl.ds(pcol, TPW)], idx0_v)
        pltpu.sync_copy(pos1_hbm.at[prow, pl.ds(pcol, TPW)], idx1_v)
        pltpu.sync_copy(p0_hbm.at[pl.ds(tb, TPW)], pb0_v)
        pltpu.sync_copy(p1_hbm.at[pl.ds(tb, TPW)], pb1_v)

        H = TPW // 2
        gathers = []
        for h in range(2):
            sl = pl.ds(h * H, H)
            gathers.append((
                pltpu.async_copy(y_hbm.at[idx0_v.at[sl]], rows0_v.at[sl],
                                 semg[h]),
                pltpu.async_copy(y_hbm.at[idx1_v.at[sl]], rows1_v.at[sl],
                                 semg[h]),
            ))

        def body(j, _):
            g0 = pb0_v[j, :]
            g1 = pb1_v[j, :]
            for c in range(D // 16):
                a = rows0_v[j, pl.ds(c * 16, 16)]
                b = rows1_v[j, pl.ds(c * 16, 16)]
                rows0_v[j, pl.ds(c * 16, 16)] = a * g0 + b * g1
            return 0

        stores = []
        for h in range(2):
            for cp in gathers[h]:
                cp.wait()
            lax.fori_loop(h * H, (h + 1) * H, body, 0)
            stores.append(pltpu.async_copy(
                rows0_v.at[pl.ds(h * H, H)],
                out_hbm.at[pl.ds(tb + h * H, H)], sem_s))
        for cp in stores:
            cp.wait()

    return _combine


def kernel(input_batch, router_w, w_in, b_in, w_out, b_out):
    orig_shape = input_batch.shape
    x = input_batch.reshape(T, D)
    pos0, pos1, p0, p1, blk = _route(x, router_w.T)
    xs = _dispatch_kernel()(x, pos0, pos1)
    y = _experts(blk, xs, w_in, b_in, w_out, b_out)
    out = _combine_kernel()(y, pos0, pos1, p0, p1)
    return out.reshape(orig_shape)


# speedup vs baseline: 1.0148x; 1.0148x over previous
import functools

import jax
import jax.numpy as jnp
from jax import lax
from jax.experimental import pallas as pl
from jax.experimental.pallas import tpu as pltpu
from jax.experimental.pallas import tpu_sc as plsc

E = 8
K = 2
D = 768
F = 1536
T = 2048
S = T * K
BM = 128
NB = S // BM + E
SPAD = NB * BM
NC, NS = 2, 16
NW = NC * NS
LPW = S // NW
TPW = T // NW

_LANES = 128
_NEG = -1e30


def _route_body(x_ref, rw_ref, pos0_ref, pos1_ref, p0_ref, p1_ref, blk_ref):
    x = x_ref[...]
    rw = rw_ref[...]
    logits = lax.dot_general(x, rw, (((1,), (1,)), ((), ())),
                             preferred_element_type=jnp.float32)
    logits = jnp.concatenate(
        [logits, jnp.full((T, 16 - E), _NEG, jnp.float32)], axis=1)
    col = lax.broadcasted_iota(jnp.int32, (T, 16), 1)
    valid = col < E
    lm = jnp.where(valid, logits, _NEG)
    mx = jnp.max(lm, axis=1, keepdims=True)
    ex = jnp.where(valid, jnp.exp(lm - mx), 0.0)
    probs = ex / jnp.sum(ex, axis=1, keepdims=True)

    p0 = jnp.max(probs, axis=1, keepdims=True)
    i0 = jnp.min(jnp.where(probs == p0, col, 127), axis=1, keepdims=True)
    probs1 = jnp.where(col == i0, 0.0, probs)
    p1 = jnp.max(probs1, axis=1, keepdims=True)
    i1 = jnp.min(jnp.where(probs1 == p1, col, 127), axis=1, keepdims=True)

    m = ((col == 2 * i0) | (col == 2 * i1 + 1)).astype(jnp.float32)

    incl = m
    sh = 1
    while sh < T:
        shifted = jnp.concatenate(
            [jnp.zeros((sh, 16), jnp.float32), incl[: T - sh, :]], axis=0)
        incl = incl + shifted
        sh *= 2
    excl = incl - m
    tot = incl[T - 1 : T, :]

    col1 = lax.broadcasted_iota(jnp.int32, (1, 16), 1)
    starts = []
    counts = []
    acc = jnp.int32(0)
    for e in range(E):
        ce = jnp.sum(jnp.where((col1 == 2 * e) | (col1 == 2 * e + 1), tot, 0.0))
        ce = ce.astype(jnp.int32)
        counts.append(ce)
        starts.append(acc)
        acc = acc + ((ce + BM - 1) // BM) * BM
    total_pad = acc

    off = jnp.zeros((1, 16), jnp.float32)
    for e in range(E):
        off = off + jnp.where((col1 == 2 * e) | (col1 == 2 * e + 1),
                              starts[e].astype(jnp.float32), 0.0)
    tot_even = jnp.where(col1 % 2 == 0, tot, 0.0)
    off = off + jnp.concatenate(
        [jnp.zeros((1, 1), jnp.float32), tot_even[:, :15]], axis=1)

    slotpos = excl + off
    pos0 = jnp.sum(jnp.where(col == 2 * i0, slotpos, 0.0), axis=1, keepdims=True)
    pos1 = jnp.sum(jnp.where(col == 2 * i1 + 1, slotpos, 0.0), axis=1, keepdims=True)
    pos0_ref[...] = jnp.reshape(pos0, (T // _LANES, _LANES)).astype(jnp.int32)
    pos1_ref[...] = jnp.reshape(pos1, (T // _LANES, _LANES)).astype(jnp.int32)
    p0_ref[...] = jnp.broadcast_to(p0, (T, 16))
    p1_ref[...] = jnp.broadcast_to(p1, (T, 16))

    kb = lax.broadcasted_iota(jnp.int32, (1, _LANES), 1)
    row_start = kb * BM
    eb = jnp.full((1, _LANES), -1, jnp.int32)
    match_start = jnp.zeros((1, _LANES), jnp.bool_)
    for e in range(E):
        eb = eb + (row_start >= starts[e]).astype(jnp.int32)
    blkrow = jnp.where(row_start < total_pad, eb, -1)
    for e in range(E):
        match_start = match_start | (row_start == starts[e])
    isfrow = ((blkrow >= 0) & match_start).astype(jnp.int32)

    active = [counts[e] > 0 for e in range(E)]
    na = jnp.int32(-1)
    next_of = [None] * E
    for e in reversed(range(E)):
        next_of[e] = na
        na = jnp.where(active[e], jnp.int32(e), na)
    rank = []
    r = jnp.int32(0)
    for e in range(E):
        rank.append(r)
        r = r + active[e].astype(jnp.int32)

    frow = jnp.full((1, _LANES), -1, jnp.int32)
    ringrow = jnp.zeros((1, _LANES), jnp.int32)
    for e in range(E):
        sel = blkrow == e
        frow = jnp.where(sel & (isfrow == 1), next_of[e], frow)
        ringrow = jnp.where(sel, rank[e] & 1, ringrow)

    blk_ref[0:1, :] = blkrow
    blk_ref[1:2, :] = isfrow
    blk_ref[2:3, :] = frow
    blk_ref[3:4, :] = ringrow
    blk_ref[4:5, :] = jnp.full((1, _LANES), 1, jnp.int32) * (total_pad // BM - 1)
    blk_ref[5:8, :] = jnp.zeros((3, _LANES), jnp.int32)


def _route(x, rw_pad):
    return pl.pallas_call(
        _route_body,
        out_shape=(
            jax.ShapeDtypeStruct((T // _LANES, _LANES), jnp.int32),
            jax.ShapeDtypeStruct((T // _LANES, _LANES), jnp.int32),
            jax.ShapeDtypeStruct((T, 16), jnp.float32),
            jax.ShapeDtypeStruct((T, 16), jnp.float32),
            jax.ShapeDtypeStruct((8, _LANES), jnp.int32),
        ),
    )(x, rw_pad)


@functools.lru_cache(maxsize=None)
def _sc_mesh():
    return plsc.VectorSubcoreMesh(
        core_axis_name="c", subcore_axis_name="s", num_cores=NC, num_subcores=NS)


_NCHUNK = 4
_CROWS = LPW // _NCHUNK


@functools.lru_cache(maxsize=None)
def _dispatch_kernel():
    @functools.partial(
        pl.kernel,
        out_type=jax.ShapeDtypeStruct((SPAD, D), jnp.float32),
        mesh=_sc_mesh(),
        scratch_types=[
            pltpu.VMEM((_NCHUNK, _CROWS), jnp.int32),
            pltpu.VMEM((LPW, D), jnp.float32),
            [pltpu.SemaphoreType.DMA] * _NCHUNK,
            pltpu.SemaphoreType.DMA,
        ],
    )
    def _dispatch(x_hbm, pos0_hbm, pos1_hbm, xs_hbm,
                  dst_v, rows_v, sems_l, sem_s):
        cid = lax.axis_index("c")
        sid = lax.axis_index("s")
        tok_base = sid * LPW

        loads = [
            pltpu.async_copy(
                x_hbm.at[pl.ds(tok_base + c * _CROWS, _CROWS)],
                rows_v.at[pl.ds(c * _CROWS, _CROWS)], sems_l[c])
            for c in range(_NCHUNK)
        ]

        for c in range(_NCHUNK):
            @pl.when(cid == 0)
            def _():
                pltpu.sync_copy(
                    pos0_hbm.at[sid, pl.ds(c * _CROWS, _CROWS)], dst_v.at[c])

            @pl.when(cid == 1)
            def _():
                pltpu.sync_copy(
                    pos1_hbm.at[sid, pl.ds(c * _CROWS, _CROWS)], dst_v.at[c])

        scats = []
        for c in range(_NCHUNK):
            loads[c].wait()
            scats.append(pltpu.async_copy(
                rows_v.at[pl.ds(c * _CROWS, _CROWS)],
                xs_hbm.at[dst_v.at[c]], sem_s))
        for cp in scats:
            cp.wait()

    return _dispatch


def _expert_body(meta_ref, xs_ref, win_hbm, bin_ref, wout_hbm, bout_ref, y_ref,
                 win_v, wout_v, semw):
    b = pl.program_id(0)
    e = meta_ref[0, b]
    isf = meta_ref[1, b]
    fe = meta_ref[2, b]
    ring = meta_ref[3, b]

    @pl.when(b == 0)
    def _():
        e0 = meta_ref[0, 0]
        pltpu.make_async_copy(win_hbm.at[e0], win_v.at[0], semw.at[0]).start()
        pltpu.make_async_copy(wout_hbm.at[e0], wout_v.at[0], semw.at[1]).start()

    @pl.when(fe >= 0)
    def _():
        nring = 1 - ring
        pltpu.make_async_copy(
            win_hbm.at[fe], win_v.at[nring], semw.at[2 * nring]).start()
        pltpu.make_async_copy(
            wout_hbm.at[fe], wout_v.at[nring], semw.at[2 * nring + 1]).start()

    @pl.when(isf == 1)
    def _():
        pltpu.make_async_copy(
            win_hbm.at[e], win_v.at[ring], semw.at[2 * ring]).wait()
        pltpu.make_async_copy(
            wout_hbm.at[e], wout_v.at[ring], semw.at[2 * ring + 1]).wait()

    @pl.when(e >= 0)
    def _():
        xb = xs_ref[...]
        h = jnp.dot(xb, win_v[ring], preferred_element_type=jnp.float32)
        h = jnp.maximum(h + bin_ref[0], 0.0)
        y = jnp.dot(h, wout_v[ring], preferred_element_type=jnp.float32)
        y_ref[...] = y + bout_ref[0]


def _experts(meta, xs, w_in, b_in, w_out, b_out):
    def eidx(b, blk):
        return jnp.where(blk[0, b] < 0, E - 1, blk[0, b])

    def bclamp(b, blk):
        return jnp.where(blk[0, b] < 0, blk[4, 0], b)

    grid_spec = pltpu.PrefetchScalarGridSpec(
        num_scalar_prefetch=1,
        grid=(NB,),
        in_specs=[
            pl.BlockSpec((BM, D), lambda b, blk: (bclamp(b, blk), 0)),
            pl.BlockSpec(memory_space=pl.ANY),
            pl.BlockSpec((1, 1, F), lambda b, blk: (eidx(b, blk), 0, 0)),
            pl.BlockSpec(memory_space=pl.ANY),
            pl.BlockSpec((1, 1, D), lambda b, blk: (eidx(b, blk), 0, 0)),
        ],
        out_specs=pl.BlockSpec((BM, D), lambda b, blk: (bclamp(b, blk), 0)),
        scratch_shapes=[
            pltpu.VMEM((2, D, F), jnp.float32),
            pltpu.VMEM((2, F, D), jnp.float32),
            pltpu.SemaphoreType.DMA((4,)),
        ],
    )
    return pl.pallas_call(
        _expert_body,
        grid_spec=grid_spec,
        out_shape=jax.ShapeDtypeStruct((SPAD, D), jnp.float32),
    )(meta, xs, w_in, b_in.reshape(E, 1, F), w_out, b_out.reshape(E, 1, D))


@functools.lru_cache(maxsize=None)
def _combine_kernel():
    @functools.partial(
        pl.kernel,
        out_type=jax.ShapeDtypeStruct((T, D), jnp.float32),
        mesh=_sc_mesh(),
        scratch_types=[
            pltpu.VMEM((TPW,), jnp.int32),
            pltpu.VMEM((TPW,), jnp.int32),
            pltpu.VMEM((TPW, 16), jnp.float32),
            pltpu.VMEM((TPW, 16), jnp.float32),
            pltpu.VMEM((TPW, D), jnp.float32),
            pltpu.VMEM((TPW, D), jnp.float32),
            [pltpu.SemaphoreType.DMA] * 2,
            pltpu.SemaphoreType.DMA,
        ],
    )
    def _combine(y_hbm, pos0_hbm, pos1_hbm, p0_hbm, p1_hbm, out_hbm,
                 idx0_v, idx1_v, pb0_v, pb1_v, rows0_v, rows1_v, semg, sem_s):
        cid = lax.axis_index("c")
        sid = lax.axis_index("s")
        wid = cid * NS + sid
        tb = wid * TPW
        prow = wid // 2
        pcol = (wid % 2) * TPW
        pltpu.sync_copy(pos0_hbm.at[prow, pl.ds(pcol, TPW)], idx0_v)
        pltpu.sync_copy(pos1_hbm.at[prow, pl.ds(pcol, TPW)], idx1_v)
        pltpu.sync_copy(p0_hbm.at[pl.ds(tb, TPW)], pb0_v)
        pltpu.sync_copy(p1_hbm.at[pl.ds(tb, TPW)], pb1_v)

        H = TPW // 2
        gathers = []
        for h in range(2):
            sl = pl.ds(h * H, H)
            gathers.append((
                pltpu.async_copy(y_hbm.at[idx0_v.at[sl]], rows0_v.at[sl],
                                 semg[h]),
                pltpu.async_copy(y_hbm.at[idx1_v.at[sl]], rows1_v.at[sl],
                                 semg[h]),
            ))

        def body(j, _):
            g0 = pb0_v[j, :]
            g1 = pb1_v[j, :]
            for c in range(D // 16):
                a = rows0_v[j, pl.ds(c * 16, 16)]
                b = rows1_v[j, pl.ds(c * 16, 16)]
                rows0_v[j, pl.ds(c * 16, 16)] = a * g0 + b * g1
            return 0

        stores = []
        for h in range(2):
            for cp in gathers[h]:
                cp.wait()
            lax.fori_loop(h * H, (h + 1) * H, body, 0)
            stores.append(pltpu.async_copy(
                rows0_v.at[pl.ds(h * H, H)],
                out_hbm.at[pl.ds(tb + h * H, H)], sem_s))
        for cp in stores:
            cp.wait()

    return _combine


def kernel(input_batch, router_w, w_in, b_in, w_out, b_out):
    orig_shape = input_batch.shape
    x = input_batch.reshape(T, D)
    pos0, pos1, p0, p1, blk = _route(x, router_w.T)
    xs = _dispatch_kernel()(x, pos0, pos1)
    y = _experts(blk, xs, w_in, b_in, w_out, b_out)
    out = _combine_kernel()(y, pos0, pos1, p0, p1)
    return out.reshape(orig_shape)
